# baseline (speedup 1.0000x reference)
import functools

import jax
import jax.numpy as jnp
from jax import lax
from jax.experimental import pallas as pl
from jax.experimental.pallas import tpu as pltpu
from jax.experimental.pallas import tpu_sc as plsc

_TOPK = 16
_N_CHUNKS = 8


def _topk_body(n_cols, k_ref, vals_ref, inds_ref, run_v, run_i):
    c = pl.program_id(1)
    nc = pl.num_programs(1)
    rb, cb = k_ref.shape
    ng = cb // 128

    @pl.when(c == 0)
    def _():
        run_v[...] = jnp.full((rb, _TOPK), -1.0, jnp.float32)
        run_i[...] = jnp.zeros((rb, _TOPK), jnp.int32)

    lane_iota = lax.broadcasted_iota(jnp.int32, (rb, 128), 1)
    lane16 = lax.broadcasted_iota(jnp.int32, (rb, _TOPK), 1)
    big = jnp.int32(2 ** 30)
    base = c * cb

    def fold(lv, lc):
        acc_v = jnp.full((rb, 128), -2.0, jnp.float32)
        acc_c = jnp.full((rb, 128), big, jnp.int32)
        for k in range(ng):
            chk = k_ref[:, k * 128:(k + 1) * 128]
            colk = base + k * 128 + lane_iota
            rem = (chk < lv) | ((chk == lv) & (colk > lc))
            v = jnp.where(rem & (colk < n_cols), chk, -2.0)
            better = v > acc_v
            acc_c = jnp.where(better, colk, acc_c)
            acc_v = jnp.where(better, v, acc_v)
        return acc_v, acc_c

    def beats(v, cidx, rv, ri):
        lastv = rv[:, _TOPK - 1:]
        lasti = ri[:, _TOPK - 1:]
        return (v > lastv) | ((v == lastv) & (cidx < lasti))

    def merge_cond(s):
        wv, wc, lv, lc, rv, ri = s
        return jnp.any(beats(wv, wc, rv, ri))

    def merge_body(s):
        wv, wc, lv, lc, rv, ri = s
        m = jnp.max(wv, axis=1)
        cstar = jnp.min(jnp.where(wv == m[:, None], wc, big), axis=1)
        upd = beats(m[:, None], cstar[:, None], rv, ri)[:, 0]
        pos = jnp.sum(((rv > m[:, None]) |
                       ((rv == m[:, None]) & (ri < cstar[:, None])))
                      .astype(jnp.int32), axis=1)[:, None]
        sh_v = jnp.roll(rv, 1, axis=1)
        sh_i = jnp.roll(ri, 1, axis=1)
        new_v = jnp.where(lane16 < pos, rv, jnp.where(lane16 == pos, m[:, None], sh_v))
        new_i = jnp.where(lane16 < pos, ri, jnp.where(lane16 == pos, cstar[:, None], sh_i))
        rv = jnp.where(upd[:, None], new_v, rv)
        ri = jnp.where(upd[:, None], new_i, ri)
        cons = (wv == m[:, None]) & (wc == cstar[:, None]) & upd[:, None]
        lv = jnp.where(cons, m[:, None], lv)
        lc = jnp.where(cons, cstar[:, None], lc)
        wv = jnp.where(cons, -2.0, wv)
        return wv, wc, lv, lc, rv, ri

    def cond(state):
        cv, cc, lv, lc, rv, ri = state
        return jnp.any(beats(cv, cc, rv, ri))

    def body(state):
        cv, cc, lv, lc, rv, ri = state
        _, _, lv, lc, rv, ri = lax.while_loop(
            merge_cond, merge_body, (cv, cc, lv, lc, rv, ri))
        cv, cc = fold(lv, lc)
        return cv, cc, lv, lc, rv, ri

    lv0 = jnp.full((rb, 128), 2.0, jnp.float32)
    lc0 = jnp.full((rb, 128), -1, jnp.int32)
    cv0, cc0 = fold(lv0, lc0)
    init = (cv0, cc0, lv0, lc0, run_v[...], run_i[...])
    _, _, _, _, rv, ri = lax.while_loop(cond, body, init)
    run_v[...] = rv
    run_i[...] = ri

    @pl.when(c == nc - 1)
    def _():
        vals_ref[...] = rv
        inds_ref[...] = ri


def _topk(K, rb=64):
    b, n = K.shape
    rb = min(rb, b)
    cb = ((n + _N_CHUNKS - 1) // _N_CHUNKS + 127) // 128 * 128
    grid = (b // rb, pl.cdiv(n, cb))
    return pl.pallas_call(
        functools.partial(_topk_body, n),
        grid=grid,
        in_specs=[pl.BlockSpec((rb, cb), lambda i, j: (i, j))],
        out_specs=[
            pl.BlockSpec((rb, _TOPK), lambda i, j: (i, 0)),
            pl.BlockSpec((rb, _TOPK), lambda i, j: (i, 0)),
        ],
        out_shape=[
            jax.ShapeDtypeStruct((b, _TOPK), jnp.float32),
            jax.ShapeDtypeStruct((b, _TOPK), jnp.int32),
        ],
        scratch_shapes=[
            pltpu.VMEM((rb, _TOPK), jnp.float32),
            pltpu.VMEM((rb, _TOPK), jnp.int32),
        ],
        compiler_params=pltpu.CompilerParams(
            dimension_semantics=("parallel", "arbitrary"),
        ),
    )(K)


def _sc_gather(table, idx_flat):
    info = plsc.get_sparse_core_info()
    nw = info.num_cores * info.num_subcores
    nb, d = idx_flat.shape[0], table.shape[1]
    b_per_w = nb // nw
    mesh = plsc.VectorSubcoreMesh(core_axis_name="c", subcore_axis_name="s")

    @functools.partial(
        pl.kernel,
        mesh=mesh,
        out_type=jax.ShapeDtypeStruct((nb, d), jnp.float32),
        scratch_types=[
            pltpu.VMEM((b_per_w,), jnp.int32),
            pltpu.VMEM((b_per_w, d), jnp.float32),
            pltpu.SemaphoreType.DMA,
        ],
    )
    def gather_k(table_hbm, idx_hbm, out_hbm, idx_v, rows_v, sem):
        wid = lax.axis_index("s") * info.num_cores + lax.axis_index("c")
        base = wid * b_per_w
        pltpu.sync_copy(idx_hbm.at[pl.ds(base, b_per_w)], idx_v)
        pltpu.async_copy(table_hbm.at[idx_v], rows_v, sem).wait()
        pltpu.sync_copy(rows_v, out_hbm.at[pl.ds(base, b_per_w)])

    return gather_k(table, idx_flat)


def _slerp_body(d, dp, w_ref, par_ref, v_ref, out_ref):
    w = w_ref[...]

    def pick(i):
        lo = v_ref[:, i * dp:i * dp + d]
        hi = v_ref[:, i * dp + d:(i + 1) * dp]
        return jnp.where(par_ref[:, i:i + 1] == 1, hi, lo)

    mu = pick(0)
    w_sum = w[:, 0:1]
    for i in range(1, _TOPK):
        vi = pick(i)
        dot = jnp.sum(mu * vi, axis=1, keepdims=True)
        dot = jnp.clip(dot, -1.0 + 1e-07, 1.0 - 1e-07)
        theta = jnp.arctan2(jnp.sqrt(1.0 - dot * dot), dot)
        wi = w[:, i:i + 1]
        w_sum = w_sum + wi
        t = wi / w_sum
        mu_p = (jnp.sin((1.0 - t) * theta) * mu + jnp.sin(t * theta) * vi) \
            / jnp.sin(theta)
        mu = jnp.where(theta == 0.0, mu, mu_p)
    out_ref[...] = mu


def _slerp(w, par, v2d, d, dp):
    b = w.shape[0]
    rb = 256
    return pl.pallas_call(
        functools.partial(_slerp_body, d, dp),
        grid=(b // rb,),
        in_specs=[
            pl.BlockSpec((rb, _TOPK), lambda i: (i, 0)),
            pl.BlockSpec((rb, _TOPK), lambda i: (i, 0)),
            pl.BlockSpec((rb, _TOPK * dp), lambda i: (i, 0)),
        ],
        out_specs=pl.BlockSpec((rb, d), lambda i: (i, 0)),
        out_shape=jax.ShapeDtypeStruct((b, d), jnp.float32),
    )(w, par, v2d)


def kernel(K, y, topk):
    b = K.shape[0]
    n, d = y.shape
    vals, inds = _topk(K)
    w = vals + jnp.asarray(topk - _TOPK, vals.dtype)
    dp = (128 // d) * d if d < 128 else d
    rows_per = dp // d
    y_wide = y.reshape(n // rows_per, dp)
    idx_flat = inds.reshape(-1)
    rows = _sc_gather(y_wide, idx_flat // rows_per)
    par = (idx_flat % rows_per).reshape(b, _TOPK)
    pre = _slerp(w, par, rows.reshape(b, _TOPK * dp), d, dp)
    return (pre, inds)


# speedup vs baseline: 1.2575x; 1.2575x over previous
import functools

import jax
import jax.numpy as jnp
from jax import lax
from jax.experimental import pallas as pl
from jax.experimental.pallas import tpu as pltpu
from jax.experimental.pallas import tpu_sc as plsc

_TOPK = 16
_N_CHUNKS = 8


def _topk_body(n_cols, has_tail, k_ref, vals_ref, inds_ref, run_v, run_i):
    c = pl.program_id(1)
    nc = pl.num_programs(1)
    rb, cb = k_ref.shape
    ng = cb // 128

    @pl.when(c == 0)
    def _():
        run_v[...] = jnp.full((rb, _TOPK), -1.0, jnp.float32)
        run_i[...] = jnp.zeros((rb, _TOPK), jnp.int32)

    lane_iota = lax.broadcasted_iota(jnp.int32, (rb, 128), 1)
    lane16 = lax.broadcasted_iota(jnp.int32, (rb, _TOPK), 1)
    big = jnp.int32(2 ** 30)
    base = c * cb

    def fold(lv, lc, masked, tail):
        acc_v = jnp.full((rb, 128), -2.0, jnp.float32)
        acc_c = jnp.full((rb, 128), big, jnp.int32)
        for k in range(ng):
            chk = k_ref[:, k * 128:(k + 1) * 128]
            colk = base + k * 128 + lane_iota
            v = chk
            if masked:
                rem = (v < lv) | ((v == lv) & (colk > lc))
                v = jnp.where(rem, v, -2.0)
            if tail:
                v = jnp.where(colk < n_cols, v, -2.0)
            better = v > acc_v
            acc_c = jnp.where(better, colk, acc_c)
            acc_v = jnp.where(better, v, acc_v)
        return acc_v, acc_c

    def beats(v, cidx, rv, ri):
        lastv = rv[:, _TOPK - 1:]
        lasti = ri[:, _TOPK - 1:]
        return (v > lastv) | ((v == lastv) & (cidx < lasti))

    def merge_cond(s):
        wv, wc, lv, lc, rv, ri = s
        return jnp.any(beats(wv, wc, rv, ri))

    def merge_body(s):
        wv, wc, lv, lc, rv, ri = s
        m = jnp.max(wv, axis=1)
        cstar = jnp.min(jnp.where(wv == m[:, None], wc, big), axis=1)
        upd = beats(m[:, None], cstar[:, None], rv, ri)[:, 0]
        pos = jnp.sum(((rv > m[:, None]) |
                       ((rv == m[:, None]) & (ri < cstar[:, None])))
                      .astype(jnp.int32), axis=1)[:, None]
        sh_v = jnp.roll(rv, 1, axis=1)
        sh_i = jnp.roll(ri, 1, axis=1)
        new_v = jnp.where(lane16 < pos, rv, jnp.where(lane16 == pos, m[:, None], sh_v))
        new_i = jnp.where(lane16 < pos, ri, jnp.where(lane16 == pos, cstar[:, None], sh_i))
        rv = jnp.where(upd[:, None], new_v, rv)
        ri = jnp.where(upd[:, None], new_i, ri)
        cons = (wv == m[:, None]) & (wc == cstar[:, None]) & upd[:, None]
        lv = jnp.where(cons, m[:, None], lv)
        lc = jnp.where(cons, cstar[:, None], lc)
        wv = jnp.where(cons, -2.0, wv)
        return wv, wc, lv, lc, rv, ri

    def cond(state):
        cv, cc, lv, lc, rv, ri = state
        return jnp.any(beats(cv, cc, rv, ri))

    def run_chunk(tail):
        def body(state):
            cv, cc, lv, lc, rv, ri = state
            _, _, lv, lc, rv, ri = lax.while_loop(
                merge_cond, merge_body, (cv, cc, lv, lc, rv, ri))
            cv, cc = fold(lv, lc, True, tail)
            return cv, cc, lv, lc, rv, ri

        lv0 = jnp.full((rb, 128), 2.0, jnp.float32)
        lc0 = jnp.full((rb, 128), -1, jnp.int32)
        cv0, cc0 = fold(lv0, lc0, False, tail)
        init = (cv0, cc0, lv0, lc0, run_v[...], run_i[...])
        _, _, _, _, rv, ri = lax.while_loop(cond, body, init)
        run_v[...] = rv
        run_i[...] = ri

    @pl.when(c < nc - 1)
    def _():
        run_chunk(False)

    @pl.when(c == nc - 1)
    def _():
        run_chunk(has_tail)
        vals_ref[...] = run_v[...]
        inds_ref[...] = run_i[...]


def _topk(K, rb=256):
    b, n = K.shape
    rb = min(rb, b)
    cb = ((n + _N_CHUNKS - 1) // _N_CHUNKS + 127) // 128 * 128
    n_cb = pl.cdiv(n, cb)
    grid = (b // rb, n_cb)
    return pl.pallas_call(
        functools.partial(_topk_body, n, n_cb * cb != n),
        grid=grid,
        in_specs=[pl.BlockSpec((rb, cb), lambda i, j: (i, j))],
        out_specs=[
            pl.BlockSpec((rb, _TOPK), lambda i, j: (i, 0)),
            pl.BlockSpec((rb, _TOPK), lambda i, j: (i, 0)),
        ],
        out_shape=[
            jax.ShapeDtypeStruct((b, _TOPK), jnp.float32),
            jax.ShapeDtypeStruct((b, _TOPK), jnp.int32),
        ],
        scratch_shapes=[
            pltpu.VMEM((rb, _TOPK), jnp.float32),
            pltpu.VMEM((rb, _TOPK), jnp.int32),
        ],
        compiler_params=pltpu.CompilerParams(
            dimension_semantics=("parallel", "arbitrary"),
        ),
    )(K)


def _sc_gather(table, idx_flat):
    info = plsc.get_sparse_core_info()
    nw = info.num_cores * info.num_subcores
    nb, d = idx_flat.shape[0], table.shape[1]
    b_per_w = nb // nw
    mesh = plsc.VectorSubcoreMesh(core_axis_name="c", subcore_axis_name="s")

    @functools.partial(
        pl.kernel,
        mesh=mesh,
        out_type=jax.ShapeDtypeStruct((nb, d), jnp.float32),
        scratch_types=[
            pltpu.VMEM((b_per_w,), jnp.int32),
            pltpu.VMEM((b_per_w, d), jnp.float32),
            pltpu.SemaphoreType.DMA,
        ],
    )
    def gather_k(table_hbm, idx_hbm, out_hbm, idx_v, rows_v, sem):
        wid = lax.axis_index("s") * info.num_cores + lax.axis_index("c")
        base = wid * b_per_w
        pltpu.sync_copy(idx_hbm.at[pl.ds(base, b_per_w)], idx_v)
        pltpu.async_copy(table_hbm.at[idx_v], rows_v, sem).wait()
        pltpu.sync_copy(rows_v, out_hbm.at[pl.ds(base, b_per_w)])

    return gather_k(table, idx_flat)


def _slerp_body(d, dp, w_ref, par_ref, v_ref, out_ref):
    w = w_ref[...]

    def pick(i):
        lo = v_ref[:, i * dp:i * dp + d]
        hi = v_ref[:, i * dp + d:(i + 1) * dp]
        return jnp.where(par_ref[:, i:i + 1] == 1, hi, lo)

    mu = pick(0)
    w_sum = w[:, 0:1]
    for i in range(1, _TOPK):
        vi = pick(i)
        dot = jnp.sum(mu * vi, axis=1, keepdims=True)
        dot = jnp.clip(dot, -1.0 + 1e-07, 1.0 - 1e-07)
        theta = jnp.arctan2(jnp.sqrt(1.0 - dot * dot), dot)
        wi = w[:, i:i + 1]
        w_sum = w_sum + wi
        t = wi / w_sum
        mu_p = (jnp.sin((1.0 - t) * theta) * mu + jnp.sin(t * theta) * vi) \
            / jnp.sin(theta)
        mu = jnp.where(theta == 0.0, mu, mu_p)
    out_ref[...] = mu


def _slerp(w, par, v2d, d, dp):
    b = w.shape[0]
    rb = 256
    return pl.pallas_call(
        functools.partial(_slerp_body, d, dp),
        grid=(b // rb,),
        in_specs=[
            pl.BlockSpec((rb, _TOPK), lambda i: (i, 0)),
            pl.BlockSpec((rb, _TOPK), lambda i: (i, 0)),
            pl.BlockSpec((rb, _TOPK * dp), lambda i: (i, 0)),
        ],
        out_specs=pl.BlockSpec((rb, d), lambda i: (i, 0)),
        out_shape=jax.ShapeDtypeStruct((b, d), jnp.float32),
    )(w, par, v2d)


def kernel(K, y, topk):
    b = K.shape[0]
    n, d = y.shape
    vals, inds = _topk(K)
    w = vals + jnp.asarray(topk - _TOPK, vals.dtype)
    dp = (128 // d) * d if d < 128 else d
    rows_per = dp // d
    y_wide = y.reshape(n // rows_per, dp)
    idx_flat = inds.reshape(-1)
    rows = _sc_gather(y_wide, idx_flat // rows_per)
    par = (idx_flat % rows_per).reshape(b, _TOPK)
    pre = _slerp(w, par, rows.reshape(b, _TOPK * dp), d, dp)
    return (pre, inds)
